# baseline (speedup 1.0000x reference)
import jax
import jax.numpy as jnp
from jax.experimental import pallas as pl
from jax.experimental.pallas import tpu as pltpu

_NBUF = 32
_LAG = _NBUF // 2
_NCH = 2


def _remix_body(gidx_ref, in_hbm, out_hbm, buf, rsem, wsem):
    n = out_hbm.shape[0] * _NCH

    def read(t, slot):
        return pltpu.make_async_copy(
            in_hbm.at[gidx_ref[t // _NCH], t % _NCH], buf.at[slot], rsem.at[slot]
        )

    def write(w, slot):
        return pltpu.make_async_copy(
            buf.at[slot], out_hbm.at[w // _NCH, w % _NCH], wsem.at[slot]
        )

    for t in range(n + _LAG):
        if t < n:
            slot = t % _NBUF
            if t >= _NBUF:
                write(t - _NBUF, slot).wait()
            read(t, slot).start()
        w = t - _LAG
        if 0 <= w < n:
            ws = w % _NBUF
            read(w, ws).wait()
            write(w, ws).start()
    for w in range(max(0, n - _NBUF), n):
        write(w, w % _NBUF).wait()


def kernel(sources):
    S, B, C, T = sources.shape
    ch = T // _NCH
    flat = sources.reshape(S * B, _NCH, ch)

    perm = jnp.argsort(jax.random.uniform(jax.random.key(42), (B,)))
    gidx = jnp.concatenate(
        [perm.astype(jnp.int32), (B + jnp.arange(B)).astype(jnp.int32)]
    )

    out = pl.pallas_call(
        _remix_body,
        grid_spec=pltpu.PrefetchScalarGridSpec(
            num_scalar_prefetch=1,
            grid=(1,),
            in_specs=[pl.BlockSpec(memory_space=pl.MemorySpace.ANY)],
            out_specs=pl.BlockSpec(memory_space=pl.MemorySpace.ANY),
            scratch_shapes=[
                pltpu.VMEM((_NBUF, ch), jnp.float32),
                pltpu.SemaphoreType.DMA((_NBUF,)),
                pltpu.SemaphoreType.DMA((_NBUF,)),
            ],
        ),
        out_shape=jax.ShapeDtypeStruct((S * B, _NCH, ch), sources.dtype),
    )(gidx, flat)
    return out.reshape(S, B, C, T)


# speedup vs baseline: 13.9953x; 13.9953x over previous
import jax
import jax.numpy as jnp
from jax.experimental import pallas as pl
from jax.experimental.pallas import tpu as pltpu

_NBUF = 16
_LAG = _NBUF // 2


def _remix_body(gidx_ref, in_hbm, out_hbm, buf, rsem, wsem):
    n = out_hbm.shape[0]

    def read(t, slot):
        return pltpu.make_async_copy(
            in_hbm.at[gidx_ref[t]], buf.at[slot], rsem.at[slot]
        )

    def write(w, slot):
        return pltpu.make_async_copy(buf.at[slot], out_hbm.at[w], wsem.at[slot])

    for t in range(n + _LAG):
        if t < n:
            slot = t % _NBUF
            if t >= _NBUF:
                write(t - _NBUF, slot).wait()
            read(t, slot).start()
        w = t - _LAG
        if 0 <= w < n:
            ws = w % _NBUF
            read(w, ws).wait()
            write(w, ws).start()
    for w in range(max(0, n - _NBUF), n):
        write(w, w % _NBUF).wait()


def kernel(sources):
    S, B, C, T = sources.shape
    flat = sources.reshape(S * B, C, T)

    perm = jnp.argsort(jax.random.uniform(jax.random.key(42), (B,)))
    gidx = jnp.concatenate(
        [perm.astype(jnp.int32), (B + jnp.arange(B)).astype(jnp.int32)]
    )

    out = pl.pallas_call(
        _remix_body,
        grid_spec=pltpu.PrefetchScalarGridSpec(
            num_scalar_prefetch=1,
            grid=(1,),
            in_specs=[pl.BlockSpec(memory_space=pl.MemorySpace.ANY)],
            out_specs=pl.BlockSpec(memory_space=pl.MemorySpace.ANY),
            scratch_shapes=[
                pltpu.VMEM((_NBUF, C, T), jnp.float32),
                pltpu.SemaphoreType.DMA((_NBUF,)),
                pltpu.SemaphoreType.DMA((_NBUF,)),
            ],
        ),
        out_shape=jax.ShapeDtypeStruct((S * B, C, T), sources.dtype),
    )(gidx, flat)
    return out.reshape(S, B, C, T)
